# baseline (speedup 1.0000x reference)
import functools

import jax
import jax.numpy as jnp
from jax import lax
from jax.experimental import pallas as pl
from jax.experimental.pallas import tpu as pltpu
from jax.experimental.pallas import tpu_sc as plsc

B, T, K, D = 16, 2048, 1024, 256
NW = 32
PPW = B * (K // 16) // NW
CR = 42


def _sc_loss(cent_hbm, units_hbm, ul_hbm, c_hbm, out_hbm,
             pan0_v, pan1_v, crows_v, units_v, ul_v, acc_v,
             sem_p0, sem_p1, sem_c):
    cid = lax.axis_index("c")
    sid = lax.axis_index("s")
    wid = sid * 2 + cid

    pltpu.sync_copy(ul_hbm, ul_v)
    lanes = lax.iota(jnp.int32, 16)
    zeros16 = jnp.zeros((16,), jnp.float32)

    def task_ids(i):
        tid = i * NW + wid
        return tid >> 6, tid & 63

    def load_units_row(b):
        uoff = pl.multiple_of(b * T, 16)
        pltpu.sync_copy(units_hbm.at[pl.ds(uoff, T)], units_v.at[pl.ds(0, T)])
        units_v[pl.ds(T, 16)] = jnp.zeros((16,), jnp.int32)

    def issue_panel(i, buf, sem):
        b, p = task_ids(jnp.minimum(i, PPW - 1))
        pltpu.async_copy(cent_hbm.at[b, :, pl.ds(p * 16, 16)], buf, sem)

    def drain_panel(buf, sem):
        pltpu.make_async_copy(
            cent_hbm.at[0, :, pl.ds(0, 16)], buf, sem).wait()

    b0, _ = task_ids(0)
    load_units_row(b0)
    issue_panel(0, pan0_v, sem_p0)

    def do_task(i, pan_v, sem, nxt_pan_v, nxt_sem, acc):
        b, p = task_ids(i)
        bprev, _ = task_ids(jnp.maximum(i - 1, 0))

        @pl.when(jnp.logical_and(i > 0, b != bprev))
        def _():
            load_units_row(b)

        lvec = plsc.load_gather(ul_v, [jnp.full((16,), b, jnp.int32)])
        big_l = jnp.max(lvec)
        k0 = p * 16
        l0 = (k0 * big_l) >> 10
        l0a = pl.multiple_of(l0 & ~7, 8)
        cp_rows = pltpu.async_copy(
            c_hbm.at[units_v.at[pl.ds(l0a, CR)]], crows_v, sem_c)
        drain_panel(pan_v, sem)
        issue_panel(i + 1, nxt_pan_v, nxt_sem)
        cp_rows.wait()

        basevec = (k0 + lanes) * big_l - (l0a << 10)

        def one(t):
            pos = basevec + t
            u = plsc.load_gather(crows_v, [pos >> 10, pos & 1023])
            return jnp.abs(pan_v[t] - u)

        main = big_l & ~7

        @plsc.parallel_loop(0, main, step=8, unroll=4, carry=zeros16)
        def tacc(t, a):
            s0 = (one(t) + one(t + 1)) + (one(t + 2) + one(t + 3))
            s1 = (one(t + 4) + one(t + 5)) + (one(t + 6) + one(t + 7))
            return a + (s0 + s1)

        tacc = lax.fori_loop(main, big_l, lambda t, a: a + one(t), tacc)
        inv = 1.0 / (lvec.astype(jnp.float32) * float(B))
        return acc + tacc * inv

    def pair_body(j, acc):
        i = j * 2
        acc = do_task(i, pan0_v, sem_p0, pan1_v, sem_p1, acc)
        acc = do_task(i + 1, pan1_v, sem_p1, pan0_v, sem_p0, acc)
        return acc

    acc = lax.fori_loop(0, PPW // 2, pair_body, zeros16)
    drain_panel(pan0_v, sem_p0)
    acc_v[...] = acc
    pltpu.sync_copy(acc_v, out_hbm.at[wid])


@jax.jit
def kernel(centroids, units, unit_lengths, C):
    mesh = plsc.VectorSubcoreMesh(core_axis_name="c", subcore_axis_name="s")
    run = functools.partial(
        pl.kernel,
        mesh=mesh,
        out_type=jax.ShapeDtypeStruct((NW, 16), jnp.float32),
        compiler_params=pltpu.CompilerParams(
            use_tc_tiling_on_sc=False, needs_layout_passes=False),
        scratch_types=[
            pltpu.VMEM((T, 16), jnp.float32),
            pltpu.VMEM((T, 16), jnp.float32),
            pltpu.VMEM((CR, K), jnp.float32),
            pltpu.VMEM((T + 16,), jnp.int32),
            pltpu.VMEM((B,), jnp.int32),
            pltpu.VMEM((16,), jnp.float32),
            pltpu.SemaphoreType.DMA,
            pltpu.SemaphoreType.DMA,
            pltpu.SemaphoreType.DMA,
        ],
    )(_sc_loss)
    out = run(centroids, units.reshape(-1), unit_lengths, C)
    return jnp.sum(out)


# speedup vs baseline: 1.0057x; 1.0057x over previous
import functools

import jax
import jax.numpy as jnp
from jax import lax
from jax.experimental import pallas as pl
from jax.experimental.pallas import tpu as pltpu
from jax.experimental.pallas import tpu_sc as plsc

B, T, K, D = 16, 2048, 1024, 256
NW = 32
PPW = B * (K // 16) // NW
CR = 42


def _sc_loss(cent_hbm, units_hbm, ul_hbm, c_hbm, out_hbm,
             pan0_v, pan1_v, crows_v, units_v, ul_v, acc_v,
             sem_p0, sem_p1, sem_c):
    cid = lax.axis_index("c")
    sid = lax.axis_index("s")
    wid = sid * 2 + cid

    pltpu.sync_copy(ul_hbm, ul_v)
    lanes = lax.iota(jnp.int32, 16)
    zeros16 = jnp.zeros((16,), jnp.float32)

    def task_ids(i):
        tid = i * NW + wid
        return tid >> 6, tid & 63

    def load_units_row(b):
        uoff = pl.multiple_of(b * T, 16)
        pltpu.sync_copy(units_hbm.at[pl.ds(uoff, T)], units_v.at[pl.ds(0, T)])
        units_v[pl.ds(T, 16)] = jnp.zeros((16,), jnp.int32)

    def issue_panel(i, buf, sem):
        b, p = task_ids(jnp.minimum(i, PPW - 1))
        pltpu.async_copy(cent_hbm.at[b, :, pl.ds(p * 16, 16)], buf, sem)

    def drain_panel(buf, sem):
        pltpu.make_async_copy(
            cent_hbm.at[0, :, pl.ds(0, 16)], buf, sem).wait()

    b0, _ = task_ids(0)
    load_units_row(b0)
    issue_panel(0, pan0_v, sem_p0)

    def do_task(i, pan_v, sem, nxt_pan_v, nxt_sem, acc):
        b, p = task_ids(i)
        bprev, _ = task_ids(jnp.maximum(i - 1, 0))

        @pl.when(jnp.logical_and(i > 0, b != bprev))
        def _():
            load_units_row(b)

        lvec = plsc.load_gather(ul_v, [jnp.full((16,), b, jnp.int32)])
        big_l = jnp.max(lvec)
        k0 = p * 16
        l0 = (k0 * big_l) >> 10
        l0a = pl.multiple_of(l0 & ~7, 8)
        cp_rows = pltpu.async_copy(
            c_hbm.at[units_v.at[pl.ds(l0a, CR)]], crows_v, sem_c)
        drain_panel(pan_v, sem)
        issue_panel(i + 1, nxt_pan_v, nxt_sem)
        cp_rows.wait()

        basevec = (k0 + lanes) * big_l - (l0a << 10)

        def one(t):
            pos = basevec + t
            u = plsc.load_gather(crows_v, [pos >> 10, pos & 1023])
            return jnp.abs(pan_v[t] - u)

        main = big_l & ~7

        @plsc.parallel_loop(0, main, step=8, unroll=2, carry=zeros16)
        def tacc(t, a):
            s0 = (one(t) + one(t + 1)) + (one(t + 2) + one(t + 3))
            s1 = (one(t + 4) + one(t + 5)) + (one(t + 6) + one(t + 7))
            return a + (s0 + s1)

        tacc = lax.fori_loop(main, big_l, lambda t, a: a + one(t), tacc)
        inv = 1.0 / (lvec.astype(jnp.float32) * float(B))
        return acc + tacc * inv

    def pair_body(j, acc):
        i = j * 2
        acc = do_task(i, pan0_v, sem_p0, pan1_v, sem_p1, acc)
        acc = do_task(i + 1, pan1_v, sem_p1, pan0_v, sem_p0, acc)
        return acc

    acc = lax.fori_loop(0, PPW // 2, pair_body, zeros16)
    drain_panel(pan0_v, sem_p0)
    acc_v[...] = acc
    pltpu.sync_copy(acc_v, out_hbm.at[wid])


@jax.jit
def kernel(centroids, units, unit_lengths, C):
    mesh = plsc.VectorSubcoreMesh(core_axis_name="c", subcore_axis_name="s")
    run = functools.partial(
        pl.kernel,
        mesh=mesh,
        out_type=jax.ShapeDtypeStruct((NW, 16), jnp.float32),
        compiler_params=pltpu.CompilerParams(
            use_tc_tiling_on_sc=False, needs_layout_passes=False),
        scratch_types=[
            pltpu.VMEM((T, 16), jnp.float32),
            pltpu.VMEM((T, 16), jnp.float32),
            pltpu.VMEM((CR, K), jnp.float32),
            pltpu.VMEM((T + 16,), jnp.int32),
            pltpu.VMEM((B,), jnp.int32),
            pltpu.VMEM((16,), jnp.float32),
            pltpu.SemaphoreType.DMA,
            pltpu.SemaphoreType.DMA,
            pltpu.SemaphoreType.DMA,
        ],
    )(_sc_loss)
    out = run(centroids, units.reshape(-1), unit_lengths, C)
    return jnp.sum(out)
